# baseline (speedup 1.0000x reference)
import functools

import jax
import jax.numpy as jnp
from jax import lax
from jax.experimental import pallas as pl
from jax.experimental.pallas import tpu as pltpu
from jax.experimental.pallas import tpu_sc as plsc

B = 16384
S_IN = 128
D = 64
OUT = 64
V = 1000000

_BLK = 2048


def _sc_gather_pairs(table_pairs, action):
    info = plsc.get_sparse_core_info()
    nc, ns = info.num_cores, info.num_subcores
    nw = nc * ns
    n_rows = B // nw

    mesh = plsc.VectorSubcoreMesh(core_axis_name="c", subcore_axis_name="s")

    @functools.partial(
        pl.kernel,
        mesh=mesh,
        out_type=jax.ShapeDtypeStruct((B, 2 * D), jnp.float32),
        scratch_types=[
            pltpu.VMEM((n_rows,), jnp.int32),
            pltpu.VMEM((n_rows, 2 * D), jnp.float32),
            pltpu.SemaphoreType.DMA,
        ],
    )
    def k(tbl_hbm, act_hbm, out_hbm, a_v, rows_v, sem):
        wid = lax.axis_index("s") * nc + lax.axis_index("c")
        base = wid * n_rows
        pltpu.sync_copy(act_hbm.at[pl.ds(base, n_rows)], a_v)

        def half(jc, carry):
            a_vec = a_v[pl.ds(jc * 16, 16)]
            a_v[pl.ds(jc * 16, 16)] = a_vec >> 1
            return carry

        lax.fori_loop(0, n_rows // 16, half, 0)
        pltpu.async_copy(tbl_hbm.at[a_v], rows_v, sem).wait()
        pltpu.sync_copy(rows_v, out_hbm.at[pl.ds(base, n_rows)])

    return k(table_pairs, action)


def _dense_body(state_ref, pair_ref, par_ref, ws_t_ref, bs_ref, wl_t_ref,
                bl_ref, out_t_ref):
    es_t = lax.dot_general(
        ws_t_ref[...], state_ref[...], (((1,), (1,)), ((), ())),
        preferred_element_type=jnp.float32) + bs_ref[...]
    p = par_ref[...]
    lo = pair_ref[:, :D]
    hi = pair_ref[:, D:]
    act = lo + (hi - lo) * p
    w1_t = wl_t_ref[:, :D]
    w2_t = wl_t_ref[:, D:]
    out_t_ref[...] = (
        jnp.dot(w1_t, es_t, preferred_element_type=jnp.float32)
        + lax.dot_general(w2_t, act, (((1,), (1,)), ((), ())),
                          preferred_element_type=jnp.float32)
        + bl_ref[...]
    )


def _tc_dense_t(state, pairs, parity, ws_t, bs_col, wl_t, bl_col):
    grid = (B // _BLK,)
    return pl.pallas_call(
        _dense_body,
        grid=grid,
        in_specs=[
            pl.BlockSpec((_BLK, S_IN), lambda i: (i, 0)),
            pl.BlockSpec((_BLK, 2 * D), lambda i: (i, 0)),
            pl.BlockSpec((_BLK, 1), lambda i: (i, 0)),
            pl.BlockSpec((D, S_IN), lambda i: (0, 0)),
            pl.BlockSpec((D, 1), lambda i: (0, 0)),
            pl.BlockSpec((OUT, 2 * D), lambda i: (0, 0)),
            pl.BlockSpec((OUT, 1), lambda i: (0, 0)),
        ],
        out_specs=pl.BlockSpec((OUT, _BLK), lambda i: (0, i)),
        out_shape=jax.ShapeDtypeStruct((OUT, B), jnp.float32),
    )(state, pairs, parity, ws_t, bs_col, wl_t, bl_col)


def kernel(state, action, W_state, b_state, action_table, W_lin, b_lin):
    table_pairs = action_table.reshape(V // 2, 2 * D)
    pairs = _sc_gather_pairs(table_pairs, action)
    parity = (action & 1).astype(jnp.float32).reshape(B, 1)
    out_t = _tc_dense_t(
        state,
        pairs,
        parity,
        jnp.transpose(W_state),
        b_state.reshape(D, 1),
        jnp.transpose(W_lin),
        b_lin.reshape(OUT, 1),
    )
    return jnp.transpose(out_t)


# speedup vs baseline: 2.6907x; 2.6907x over previous
import functools

import jax
import jax.numpy as jnp
from jax import lax
from jax.experimental import pallas as pl
from jax.experimental.pallas import tpu as pltpu
from jax.experimental.pallas import tpu_sc as plsc

B = 16384
S_IN = 128
D = 64
OUT = 64
V = 1000000

V_PAD = 1000064
NW = 32
WR = 31232
CH = 512
NCH = 63
GRP = 128
SEG = 512

_BLK = 2048


def _sc_gather(table_t, action):
    mesh = plsc.VectorSubcoreMesh(core_axis_name="c", subcore_axis_name="s")

    @functools.partial(
        pl.kernel,
        mesh=mesh,
        out_type=jax.ShapeDtypeStruct((B, 2 * D), jnp.float32),
        scratch_types=[
            pltpu.VMEM((B,), jnp.int32),
            pltpu.VMEM((B,), jnp.int32),
            pltpu.VMEM((B,), jnp.int32),
            pltpu.VMEM((D, CH), jnp.float32),
            pltpu.VMEM((SEG + 16,), jnp.int32),
            pltpu.VMEM((SEG + 16,), jnp.int32),
            pltpu.VMEM((GRP, 2 * D), jnp.float32),
            pltpu.VMEM((GRP,), jnp.int32),
            pltpu.SemaphoreType.DMA,
        ],
        compiler_params=pltpu.CompilerParams(needs_layout_passes=False),
    )
    def k(tbl_hbm, act_hbm, out_hbm, a_all, jl_v, vl_v, chunk_v,
          mc_v, mj_v, stage_v, sidx_v, sem):
        wid = lax.axis_index("s") * 2 + lax.axis_index("c")
        start = wid * WR
        end = jnp.where(wid == NW - 1, V, start + WR)

        pltpu.sync_copy(act_hbm, a_all)
        d16 = lax.iota(jnp.int32, 16)
        for i in range(GRP // 16):
            sidx_v[pl.ds(i * 16, 16)] = jnp.full((16,), -1, jnp.int32)

        def bucket(bc, off):
            a_vec = a_all[pl.ds(bc * 16, 16)]
            m = (a_vec >= start) & (a_vec < end)
            j_vec = lax.iota(jnp.int32, 16) + bc * 16
            plsc.store_compressed(jl_v.at[pl.ds(off, 16)], j_vec, mask=m)
            plsc.store_compressed(vl_v.at[pl.ds(off, 16)], a_vec, mask=m)
            return off + jnp.sum(m.astype(jnp.int32))

        nloc = lax.fori_loop(0, B // 16, bucket, 0)
        nseg = (nloc + SEG - 1) // SEG

        def flush():
            pltpu.async_copy(
                stage_v,
                out_hbm.at[plsc.Indices(sidx_v, ignored_value=-1)],
                sem,
            ).wait()

        def chunk_step(ci, slot):
            rs = start + ci * CH
            re = jnp.minimum(rs + CH, V)
            vb = jnp.minimum(rs, V_PAD - CH)

            def do_chunk(slot_in):
                pltpu.sync_copy(tbl_hbm.at[:, pl.ds(pl.multiple_of(vb, 128),
                                                    CH)], chunk_v)

                def seg_step(si, slot_c):
                    seg_base = si * SEG
                    seg_len = jnp.minimum(nloc - seg_base, SEG)

                    def scan(li, cnt):
                        v_vec = vl_v[pl.ds(seg_base + li * 16, 16)]
                        j_vec = jl_v[pl.ds(seg_base + li * 16, 16)]
                        in_seg = (lax.iota(jnp.int32, 16) + li * 16) < seg_len
                        m = (v_vec >= rs) & (v_vec < re) & in_seg
                        plsc.store_compressed(mc_v.at[pl.ds(cnt, 16)],
                                              v_vec - vb, mask=m)
                        plsc.store_compressed(mj_v.at[pl.ds(cnt, 16)],
                                              j_vec, mask=m)
                        return cnt + jnp.sum(m.astype(jnp.int32))

                    nmatch = lax.fori_loop(0, (SEG + 15) // 16, scan, 0)
                    ngrp = (nmatch + 15) // 16

                    def grp(g, slot_g):
                        cols = mc_v[pl.ds(g * 16, 16)]
                        js = mj_v[pl.ds(g * 16, 16)]
                        valid = (d16 + g * 16) < nmatch
                        cols = jnp.where(valid, cols, 0)
                        rowv = slot_g + d16
                        for d in range(D):
                            dv = jnp.full((16,), d, jnp.int32)
                            vals = plsc.load_gather(chunk_v, [dv, cols])
                            plsc.store_scatter(stage_v, [rowv, dv], vals,
                                               mask=valid)
                        plsc.store_scatter(sidx_v, [rowv], js, mask=valid)
                        slot_n = slot_g + jnp.sum(valid.astype(jnp.int32))

                        @pl.when(slot_n > GRP - 16)
                        def _():
                            flush()

                        return jnp.where(slot_n > GRP - 16, 0, slot_n)

                    return lax.fori_loop(0, ngrp, grp, slot_c)

                return lax.fori_loop(0, nseg, seg_step, slot_in)

            return lax.cond(rs < end, do_chunk, lambda s: s, slot)

        lax.fori_loop(0, NCH, chunk_step, 0)
        flush()

    return k(table_t, action)


def _dense_body(state_ref, pair_ref, ws_t_ref, bs_ref, wl_t_ref,
                bl_ref, out_t_ref):
    es_t = lax.dot_general(
        ws_t_ref[...], state_ref[...], (((1,), (1,)), ((), ())),
        preferred_element_type=jnp.float32) + bs_ref[...]
    act = pair_ref[:, :D]
    w1_t = wl_t_ref[:, :D]
    w2_t = wl_t_ref[:, D:]
    out_t_ref[...] = (
        jnp.dot(w1_t, es_t, preferred_element_type=jnp.float32)
        + lax.dot_general(w2_t, act, (((1,), (1,)), ((), ())),
                          preferred_element_type=jnp.float32)
        + bl_ref[...]
    )


def _tc_dense_t(state, pairs, ws_t, bs_col, wl_t, bl_col):
    grid = (B // _BLK,)
    return pl.pallas_call(
        _dense_body,
        grid=grid,
        in_specs=[
            pl.BlockSpec((_BLK, S_IN), lambda i: (i, 0)),
            pl.BlockSpec((_BLK, 2 * D), lambda i: (i, 0)),
            pl.BlockSpec((D, S_IN), lambda i: (0, 0)),
            pl.BlockSpec((D, 1), lambda i: (0, 0)),
            pl.BlockSpec((OUT, 2 * D), lambda i: (0, 0)),
            pl.BlockSpec((OUT, 1), lambda i: (0, 0)),
        ],
        out_specs=pl.BlockSpec((OUT, _BLK), lambda i: (0, i)),
        out_shape=jax.ShapeDtypeStruct((OUT, B), jnp.float32),
    )(state, pairs, ws_t, bs_col, wl_t, bl_col)


def kernel(state, action, W_state, b_state, action_table, W_lin, b_lin):
    table_t = jnp.transpose(action_table)
    pairs = _sc_gather(table_t, action)
    out_t = _tc_dense_t(
        state,
        pairs,
        jnp.transpose(W_state),
        b_state.reshape(D, 1),
        jnp.transpose(W_lin),
        b_lin.reshape(OUT, 1),
    )
    return jnp.transpose(out_t)


# speedup vs baseline: 2.8563x; 1.0616x over previous
import functools

import jax
import jax.numpy as jnp
from jax import lax
from jax.experimental import pallas as pl
from jax.experimental.pallas import tpu as pltpu
from jax.experimental.pallas import tpu_sc as plsc

B = 16384
S_IN = 128
D = 64
OUT = 64
V = 1000000

V_PAD = 1000064
NW = 32
WR = 31232
CH = 512
NCH = 63
GRP = 96
SEG = 512

_BLK = 2048


def _sc_gather(table_t, action):
    mesh = plsc.VectorSubcoreMesh(core_axis_name="c", subcore_axis_name="s")

    @functools.partial(
        pl.kernel,
        mesh=mesh,
        out_type=jax.ShapeDtypeStruct((B, 2 * D), jnp.float32),
        scratch_types=[
            pltpu.VMEM((B,), jnp.int32),
            pltpu.VMEM((B,), jnp.int32),
            pltpu.VMEM((B,), jnp.int32),
            pltpu.VMEM((D, CH), jnp.float32),
            pltpu.VMEM((D, CH), jnp.float32),
            pltpu.VMEM((SEG + 16,), jnp.int32),
            pltpu.VMEM((SEG + 16,), jnp.int32),
            pltpu.VMEM((GRP, 2 * D), jnp.float32),
            pltpu.VMEM((GRP,), jnp.int32),
            pltpu.SemaphoreType.DMA,
            pltpu.SemaphoreType.DMA,
            pltpu.SemaphoreType.DMA,
        ],
        compiler_params=pltpu.CompilerParams(needs_layout_passes=False),
    )
    def k(tbl_hbm, act_hbm, out_hbm, a_all, jl_v, vl_v, chunk_a, chunk_b,
          mc_v, mj_v, stage_v, sidx_v, sem_a, sem_b, sem_s):
        wid = lax.axis_index("s") * 2 + lax.axis_index("c")
        start = wid * WR
        end = jnp.where(wid == NW - 1, V, start + WR)

        pltpu.sync_copy(act_hbm, a_all)
        d16 = lax.iota(jnp.int32, 16)
        for i in range(GRP // 16):
            sidx_v[pl.ds(i * 16, 16)] = jnp.full((16,), -1, jnp.int32)

        def bucket(bc, off):
            a_vec = a_all[pl.ds(bc * 16, 16)]
            m = (a_vec >= start) & (a_vec < end)
            j_vec = lax.iota(jnp.int32, 16) + bc * 16
            plsc.store_compressed(jl_v.at[pl.ds(off, 16)], j_vec, mask=m)
            plsc.store_compressed(vl_v.at[pl.ds(off, 16)], a_vec, mask=m)
            return off + jnp.sum(m.astype(jnp.int32))

        nloc = lax.fori_loop(0, B // 16, bucket, 0)
        nseg = (nloc + SEG - 1) // SEG

        def flush():
            pltpu.async_copy(
                stage_v,
                out_hbm.at[plsc.Indices(sidx_v, ignored_value=-1)],
                sem_s,
            ).wait()

        def chunk_src(ci):
            rs = start + ci * CH
            vb = jnp.minimum(rs, V_PAD - CH)
            return tbl_hbm.at[:, pl.ds(pl.multiple_of(vb, 128), CH)]

        def extract(ci, buf, slot_in):
            rs = start + ci * CH
            re = jnp.minimum(rs + CH, V)
            vb = jnp.minimum(rs, V_PAD - CH)

            def seg_step(si, slot_c):
                seg_base = si * SEG
                seg_len = jnp.minimum(nloc - seg_base, SEG)

                def scan(li, cnt):
                    v_vec = vl_v[pl.ds(seg_base + li * 16, 16)]
                    j_vec = jl_v[pl.ds(seg_base + li * 16, 16)]
                    in_seg = (lax.iota(jnp.int32, 16) + li * 16) < seg_len
                    m = (v_vec >= rs) & (v_vec < re) & in_seg
                    plsc.store_compressed(mc_v.at[pl.ds(cnt, 16)],
                                          v_vec - vb, mask=m)
                    plsc.store_compressed(mj_v.at[pl.ds(cnt, 16)],
                                          j_vec, mask=m)
                    return cnt + jnp.sum(m.astype(jnp.int32))

                nmatch = lax.fori_loop(0, (SEG + 15) // 16, scan, 0)
                ngrp = (nmatch + 15) // 16

                def grp(g, slot_g):
                    cols = mc_v[pl.ds(g * 16, 16)]
                    js = mj_v[pl.ds(g * 16, 16)]
                    valid = (d16 + g * 16) < nmatch
                    cols = jnp.where(valid, cols, 0)
                    rowv = slot_g + d16
                    for d in range(D):
                        dv = jnp.full((16,), d, jnp.int32)
                        vals = plsc.load_gather(buf, [dv, cols])
                        plsc.store_scatter(stage_v, [rowv, dv], vals,
                                           mask=valid)
                    plsc.store_scatter(sidx_v, [rowv], js, mask=valid)
                    slot_n = slot_g + jnp.sum(valid.astype(jnp.int32))

                    @pl.when(slot_n > GRP - 16)
                    def _():
                        flush()

                    return jnp.where(slot_n > GRP - 16, 0, slot_n)

                return lax.fori_loop(0, ngrp, grp, slot_c)

            return lax.fori_loop(0, nseg, seg_step, slot_in)

        def pair_step(pi, slot):
            ci0 = 2 * pi
            ci1 = 2 * pi + 1

            def with0(slot_0):
                h0 = pltpu.make_async_copy(chunk_src(ci0), chunk_a, sem_a)
                h0.start()

                def with1(slot_1):
                    h1 = pltpu.make_async_copy(chunk_src(ci1), chunk_b, sem_b)
                    h1.start()
                    h0.wait()
                    h1.wait()
                    s2 = extract(ci0, chunk_a, slot_1)
                    return extract(ci1, chunk_b, s2)

                def only0(slot_1):
                    h0.wait()
                    return extract(ci0, chunk_a, slot_1)

                return lax.cond(start + ci1 * CH < end, with1, only0, slot_0)

            return lax.cond(start + ci0 * CH < end, with0, lambda x: x, slot)

        lax.fori_loop(0, NCH // 2, pair_step, 0)
        flush()

    return k(table_t, action)


def _dense_body(state_ref, pair_ref, ws_t_ref, bs_ref, wl_t_ref,
                bl_ref, out_t_ref):
    es_t = lax.dot_general(
        ws_t_ref[...], state_ref[...], (((1,), (1,)), ((), ())),
        preferred_element_type=jnp.float32) + bs_ref[...]
    act = pair_ref[:, :D]
    w1_t = wl_t_ref[:, :D]
    w2_t = wl_t_ref[:, D:]
    out_t_ref[...] = (
        jnp.dot(w1_t, es_t, preferred_element_type=jnp.float32)
        + lax.dot_general(w2_t, act, (((1,), (1,)), ((), ())),
                          preferred_element_type=jnp.float32)
        + bl_ref[...]
    )


def _tc_dense_t(state, pairs, ws_t, bs_col, wl_t, bl_col):
    grid = (B // _BLK,)
    return pl.pallas_call(
        _dense_body,
        grid=grid,
        in_specs=[
            pl.BlockSpec((_BLK, S_IN), lambda i: (i, 0)),
            pl.BlockSpec((_BLK, 2 * D), lambda i: (i, 0)),
            pl.BlockSpec((D, S_IN), lambda i: (0, 0)),
            pl.BlockSpec((D, 1), lambda i: (0, 0)),
            pl.BlockSpec((OUT, 2 * D), lambda i: (0, 0)),
            pl.BlockSpec((OUT, 1), lambda i: (0, 0)),
        ],
        out_specs=pl.BlockSpec((OUT, _BLK), lambda i: (0, i)),
        out_shape=jax.ShapeDtypeStruct((OUT, B), jnp.float32),
    )(state, pairs, ws_t, bs_col, wl_t, bl_col)


def kernel(state, action, W_state, b_state, action_table, W_lin, b_lin):
    table_t = jnp.transpose(action_table)
    pairs = _sc_gather(table_t, action)
    out_t = _tc_dense_t(
        state,
        pairs,
        jnp.transpose(W_state),
        b_state.reshape(D, 1),
        jnp.transpose(W_lin),
        b_lin.reshape(OUT, 1),
    )
    return jnp.transpose(out_t)


# speedup vs baseline: 2.8633x; 1.0025x over previous
import functools

import jax
import jax.numpy as jnp
from jax import lax
from jax.experimental import pallas as pl
from jax.experimental.pallas import tpu as pltpu
from jax.experimental.pallas import tpu_sc as plsc

B = 16384
S_IN = 128
D = 64
OUT = 64
V = 1000000

V_PAD = 1000064
NW = 32
WR = 31232
CH = 512
NCH = 63
GRP = 96
SEG = 512

_BLK = 2048


def _sc_gather(table_t, action):
    mesh = plsc.VectorSubcoreMesh(core_axis_name="c", subcore_axis_name="s")

    @functools.partial(
        pl.kernel,
        mesh=mesh,
        out_type=jax.ShapeDtypeStruct((B, 2 * D), jnp.float32),
        scratch_types=[
            pltpu.VMEM((B // 256, 256), jnp.int32),
            pltpu.VMEM((B,), jnp.int32),
            pltpu.VMEM((B,), jnp.int32),
            pltpu.VMEM((D, CH), jnp.float32),
            pltpu.VMEM((D, CH), jnp.float32),
            pltpu.VMEM((SEG + 16,), jnp.int32),
            pltpu.VMEM((SEG + 16,), jnp.int32),
            pltpu.VMEM((GRP, 2 * D), jnp.float32),
            pltpu.VMEM((GRP,), jnp.int32),
            pltpu.SemaphoreType.DMA,
            pltpu.SemaphoreType.DMA,
            pltpu.SemaphoreType.DMA,
        ],
        compiler_params=pltpu.CompilerParams(needs_layout_passes=False),
    )
    def k(tbl_hbm, act_hbm, out_hbm, a_all, jl_v, vl_v, chunk_a, chunk_b,
          mc_v, mj_v, stage_v, sidx_v, sem_a, sem_b, sem_s):
        wid = lax.axis_index("s") * 2 + lax.axis_index("c")
        start = wid * WR
        end = jnp.where(wid == NW - 1, V, start + WR)

        pltpu.sync_copy(act_hbm, a_all)
        d16 = lax.iota(jnp.int32, 16)

        for i in range(GRP // 16):
            sidx_v[pl.ds(i * 16, 16)] = jnp.full((16,), -1, jnp.int32)

        def bucket(bc, off):
            r = bc >> 4
            c = bc & 15
            a_vec = a_all[r, pl.ds(c * 16, 16)]
            m = (a_vec >= start) & (a_vec < end)
            j_vec = lax.iota(jnp.int32, 16) + bc * 16
            plsc.store_compressed(jl_v.at[pl.ds(off, 16)], j_vec, mask=m)
            plsc.store_compressed(vl_v.at[pl.ds(off, 16)], a_vec, mask=m)
            return off + jnp.sum(m.astype(jnp.int32))

        nloc = lax.fori_loop(0, B // 16, bucket, 0)
        nseg = (nloc + SEG - 1) // SEG

        def flush():
            pltpu.async_copy(
                stage_v,
                out_hbm.at[plsc.Indices(sidx_v, ignored_value=-1)],
                sem_s,
            ).wait()

        def chunk_src(ci):
            rs = start + ci * CH
            vb = jnp.minimum(rs, V_PAD - CH)
            return tbl_hbm.at[:, pl.ds(pl.multiple_of(vb, 128), CH)]

        def extract(ci, buf, slot_in):
            rs = start + ci * CH
            re = jnp.minimum(rs + CH, V)
            vb = jnp.minimum(rs, V_PAD - CH)

            def seg_step(si, slot_c):
                seg_base = si * SEG
                seg_len = jnp.minimum(nloc - seg_base, SEG)

                def scan(li, cnt):
                    v_vec = vl_v[pl.ds(seg_base + li * 16, 16)]
                    j_vec = jl_v[pl.ds(seg_base + li * 16, 16)]
                    in_seg = (lax.iota(jnp.int32, 16) + li * 16) < seg_len
                    m = (v_vec >= rs) & (v_vec < re) & in_seg
                    plsc.store_compressed(mc_v.at[pl.ds(cnt, 16)],
                                          v_vec - vb, mask=m)
                    plsc.store_compressed(mj_v.at[pl.ds(cnt, 16)],
                                          j_vec, mask=m)
                    return cnt + jnp.sum(m.astype(jnp.int32))

                nmatch = lax.fori_loop(0, (SEG + 15) // 16, scan, 0)
                ngrp = (nmatch + 15) // 16

                def grp(g, slot_g):
                    cols = mc_v[pl.ds(g * 16, 16)]
                    js = mj_v[pl.ds(g * 16, 16)]
                    valid = (d16 + g * 16) < nmatch
                    cols = jnp.where(valid, cols, 0)
                    rowv = slot_g + d16
                    for d in range(D):
                        dv = jnp.full((16,), d, jnp.int32)
                        vals = plsc.load_gather(buf, [dv, cols])
                        plsc.store_scatter(stage_v, [rowv, dv], vals,
                                           mask=valid)
                    plsc.store_scatter(sidx_v, [rowv], js, mask=valid)
                    slot_n = slot_g + jnp.sum(valid.astype(jnp.int32))

                    @pl.when(slot_n > GRP - 16)
                    def _():
                        flush()

                    return jnp.where(slot_n > GRP - 16, 0, slot_n)

                return lax.fori_loop(0, ngrp, grp, slot_c)

            return lax.fori_loop(0, nseg, seg_step, slot_in)

        def pair_step(pi, slot):
            ci0 = 2 * pi
            ci1 = 2 * pi + 1

            def with0(slot_0):
                h0 = pltpu.make_async_copy(chunk_src(ci0), chunk_a, sem_a)
                h0.start()

                def with1(slot_1):
                    h1 = pltpu.make_async_copy(chunk_src(ci1), chunk_b, sem_b)
                    h1.start()
                    h0.wait()
                    h1.wait()
                    s2 = extract(ci0, chunk_a, slot_1)
                    return extract(ci1, chunk_b, s2)

                def only0(slot_1):
                    h0.wait()
                    return extract(ci0, chunk_a, slot_1)

                return lax.cond(start + ci1 * CH < end, with1, only0, slot_0)

            return lax.cond(start + ci0 * CH < end, with0, lambda x: x, slot)

        lax.fori_loop(0, NCH // 2, pair_step, 0)
        flush()

    return k(table_t, action)


def _dense_body(state_ref, pair_ref, ws_t_ref, bs_ref, wl_t_ref,
                bl_ref, out_t_ref):
    es_t = lax.dot_general(
        ws_t_ref[...], state_ref[...], (((1,), (1,)), ((), ())),
        preferred_element_type=jnp.float32) + bs_ref[...]
    act = pair_ref[:, :D]
    w1_t = wl_t_ref[:, :D]
    w2_t = wl_t_ref[:, D:]
    out_t_ref[...] = (
        jnp.dot(w1_t, es_t, preferred_element_type=jnp.float32)
        + lax.dot_general(w2_t, act, (((1,), (1,)), ((), ())),
                          preferred_element_type=jnp.float32)
        + bl_ref[...]
    )


def _tc_dense_t(state, pairs, ws_t, bs_col, wl_t, bl_col):
    grid = (B // _BLK,)
    return pl.pallas_call(
        _dense_body,
        grid=grid,
        in_specs=[
            pl.BlockSpec((_BLK, S_IN), lambda i: (i, 0)),
            pl.BlockSpec((_BLK, 2 * D), lambda i: (i, 0)),
            pl.BlockSpec((D, S_IN), lambda i: (0, 0)),
            pl.BlockSpec((D, 1), lambda i: (0, 0)),
            pl.BlockSpec((OUT, 2 * D), lambda i: (0, 0)),
            pl.BlockSpec((OUT, 1), lambda i: (0, 0)),
        ],
        out_specs=pl.BlockSpec((OUT, _BLK), lambda i: (0, i)),
        out_shape=jax.ShapeDtypeStruct((OUT, B), jnp.float32),
    )(state, pairs, ws_t, bs_col, wl_t, bl_col)


def kernel(state, action, W_state, b_state, action_table, W_lin, b_lin):
    table_t = jnp.transpose(action_table)
    pairs = _sc_gather(table_t, action.reshape(B // 256, 256))
    out_t = _tc_dense_t(
        state,
        pairs,
        jnp.transpose(W_state),
        b_state.reshape(D, 1),
        jnp.transpose(W_lin),
        b_lin.reshape(OUT, 1),
    )
    return jnp.transpose(out_t)
